# baseline (speedup 1.0000x reference)
import functools

import jax
import jax.numpy as jnp
from jax import lax
from jax.experimental import pallas as pl
from jax.experimental.pallas import tpu as pltpu
from jax.experimental.pallas import tpu_sc as plsc

E = 8
K = 2
H = 1024
FF = 4096
BN = 256

NC = 2
NS = 16
NW = NC * NS


def _router_body(x_ref, w_ref, b_ref, idx_ref, loss_ref, xcp_ref):
    x = x_ref[...]
    xcp_ref[...] = x
    logits = jnp.dot(x, w_ref[...], preferred_element_type=jnp.float32)
    logits = logits + b_ref[...]
    z = logits - jnp.max(logits, axis=1, keepdims=True)
    ez = jnp.exp(z)
    p = ez / jnp.sum(ez, axis=1, keepdims=True)

    iota = lax.broadcasted_iota(jnp.int32, p.shape, 1)
    m1 = jnp.max(p, axis=1, keepdims=True)
    i1 = jnp.min(jnp.where(p >= m1, iota, E), axis=1, keepdims=True)
    p2 = jnp.where(iota == i1, -jnp.float32(jnp.inf), p)
    m2 = jnp.max(p2, axis=1, keepdims=True)
    i2 = jnp.min(jnp.where(p2 >= m2, iota, E), axis=1, keepdims=True)
    idx_ref[...] = jnp.concatenate([i1, i2], axis=1)

    imp = jnp.mean(p, axis=0, keepdims=True)
    loss_ref[...] = 0.01 * jnp.sum(E * imp * jnp.log(imp)).reshape(1, 1)


def _run_router(x2d, router_w, router_b):
    n, h = x2d.shape
    top2, loss, xcp = pl.pallas_call(
        _router_body,
        out_shape=(
            jax.ShapeDtypeStruct((n, K), jnp.int32),
            jax.ShapeDtypeStruct((1, 1), jnp.float32),
            jax.ShapeDtypeStruct((n, h), jnp.float32),
        ),
    )(x2d, router_w, router_b.reshape(1, E))
    return top2, loss[0, 0], xcp


def _dispatch_plan(top2, n_tokens):
    nk = n_tokens * K
    p_total = nk + E * BN
    nb = p_total // BN

    e_flat = top2.reshape(-1).astype(jnp.int32)
    onehot = (e_flat[:, None] == jnp.arange(E, dtype=jnp.int32)[None, :])
    inc = jnp.cumsum(onehot.astype(jnp.int32), axis=0)
    rank = jnp.take_along_axis(inc, e_flat[:, None], axis=1)[:, 0] - 1
    counts = inc[-1]
    padded = ((counts + BN - 1) // BN) * BN
    pstart = jnp.concatenate(
        [jnp.zeros((1,), jnp.int32), jnp.cumsum(padded)[:-1].astype(jnp.int32)])
    pos = pstart[e_flat] + rank

    token_of_entry = (jnp.arange(nk, dtype=jnp.int32) // K)
    gather_idx = jnp.zeros((p_total,), jnp.int32).at[pos].set(token_of_entry)

    pend = (pstart + padded).astype(jnp.int32)
    blk = jnp.arange(nb, dtype=jnp.int32) * BN
    block_expert = jnp.sum((blk[:, None] >= pend[None, :]).astype(jnp.int32),
                           axis=1)
    block_expert = jnp.clip(block_expert, 0, E - 1)
    n_active = (jnp.sum(padded) // BN).astype(jnp.int32).reshape(1)

    pos_a = pos[0::K]
    pos_b = pos[1::K]
    return gather_idx, block_expert, n_active, pos_a, pos_b, p_total, nb


def _make_sc_gather(n_rows_out, d, chunk, dtype):
    rows_per_w = n_rows_out // NW
    n_chunks = rows_per_w // chunk
    mesh = plsc.VectorSubcoreMesh(core_axis_name="c", subcore_axis_name="s")

    @functools.partial(
        pl.kernel,
        mesh=mesh,
        out_type=jax.ShapeDtypeStruct((n_rows_out, d), dtype),
        scratch_types=[
            pltpu.VMEM((n_chunks, chunk), jnp.int32),
            pltpu.VMEM((chunk, d), dtype),
            pltpu.VMEM((chunk, d), dtype),
            pltpu.SemaphoreType.DMA,
            pltpu.SemaphoreType.DMA,
            pltpu.SemaphoreType.DMA,
            pltpu.SemaphoreType.DMA,
        ],
    )
    def gather(table_hbm, idx_hbm, out_hbm, idx_v, buf0, buf1,
               gs0, gs1, os0, os1):
        wid = lax.axis_index("s") * NC + lax.axis_index("c")
        base = wid * rows_per_w
        pltpu.sync_copy(idx_hbm.at[wid], idx_v)
        bufs = (buf0, buf1)
        gsems = (gs0, gs1)
        osems = (os0, os1)
        ocopies = [None, None]
        gcopies = [None, None]
        gcopies[0] = pltpu.async_copy(table_hbm.at[idx_v.at[0]], buf0, gs0)
        for c in range(n_chunks):
            p = c % 2
            q = (c + 1) % 2
            if c + 1 < n_chunks:
                if c >= 1:
                    ocopies[q].wait()
                gcopies[q] = pltpu.async_copy(
                    table_hbm.at[idx_v.at[c + 1]], bufs[q], gsems[q])
            gcopies[p].wait()
            ocopies[p] = pltpu.async_copy(
                bufs[p], out_hbm.at[pl.ds(base + c * chunk, chunk)], osems[p])
        for c in range(max(0, n_chunks - 2), n_chunks):
            ocopies[c % 2].wait()

    return gather


def _make_sc_combine(n_tokens, d, chunk):
    rows_per_w = n_tokens // NW
    n_chunks = rows_per_w // chunk
    mesh = plsc.VectorSubcoreMesh(core_axis_name="c", subcore_axis_name="s")
    lanes_per_row = d // 16

    @functools.partial(
        pl.kernel,
        mesh=mesh,
        out_type=jax.ShapeDtypeStruct((n_tokens, d), jnp.float32),
        scratch_types=[
            pltpu.VMEM((n_chunks, chunk), jnp.int32),
            pltpu.VMEM((n_chunks, chunk), jnp.int32),
            pltpu.VMEM((chunk, d), jnp.float32),
            pltpu.VMEM((chunk, d), jnp.float32),
            pltpu.VMEM((chunk, d), jnp.float32),
            pltpu.VMEM((chunk, d), jnp.float32),
            pltpu.SemaphoreType.DMA,
            pltpu.SemaphoreType.DMA,
            pltpu.SemaphoreType.DMA,
            pltpu.SemaphoreType.DMA,
        ],
    )
    def combine(y_hbm, ia_hbm, ib_hbm, out_hbm,
                ia_v, ib_v, ra0, ra1, rb0, rb1, gs0, gs1, os0, os1):
        wid = lax.axis_index("s") * NC + lax.axis_index("c")
        base = wid * rows_per_w
        pltpu.sync_copy(ia_hbm.at[wid], ia_v)
        pltpu.sync_copy(ib_hbm.at[wid], ib_v)
        ras = (ra0, ra1)
        rbs = (rb0, rb1)
        gsems = (gs0, gs1)
        osems = (os0, os1)
        ocopies = [None, None]
        gcopies = [None, None]
        gcopies[0] = (pltpu.async_copy(y_hbm.at[ia_v.at[0]], ra0, gs0),
                      pltpu.async_copy(y_hbm.at[ib_v.at[0]], rb0, gs0))
        for c in range(n_chunks):
            p = c % 2
            q = (c + 1) % 2
            if c + 1 < n_chunks:
                if c >= 1:
                    ocopies[q].wait()
                gcopies[q] = (
                    pltpu.async_copy(y_hbm.at[ia_v.at[c + 1]], ras[q], gsems[q]),
                    pltpu.async_copy(y_hbm.at[ib_v.at[c + 1]], rbs[q], gsems[q]))
            gcopies[p][0].wait()
            gcopies[p][1].wait()

            def row_add(r, carry, _ra=ras[p], _rb=rbs[p]):
                for s in range(lanes_per_row):
                    a = _ra[r, pl.ds(s * 16, 16)]
                    b = _rb[r, pl.ds(s * 16, 16)]
                    _ra[r, pl.ds(s * 16, 16)] = a + b
                return carry

            lax.fori_loop(0, chunk, row_add, 0)
            ocopies[p] = pltpu.async_copy(
                ras[p], out_hbm.at[pl.ds(base + c * chunk, chunk)], osems[p])
        for c in range(max(0, n_chunks - 2), n_chunks):
            ocopies[c % 2].wait()

    return combine


def _ffn_body(nact_ref, be_ref, x_ref, w1_ref, b1_ref, w2_ref, b2_ref, o_ref):
    i = pl.program_id(0)

    @pl.when(i < nact_ref[0])
    def _():
        x = x_ref[...].astype(jnp.bfloat16)
        h = jnp.dot(x, w1_ref[0], preferred_element_type=jnp.float32)
        h = h + b1_ref[0]
        h = h * 0.5 * (1.0 + lax.erf(h * 0.7071067811865476))
        y = jnp.dot(h.astype(jnp.bfloat16), w2_ref[0],
                    preferred_element_type=jnp.float32)
        o_ref[...] = y + b2_ref[0]


def _run_ffn(xs, fc1_w, fc1_b, fc2_w, fc2_b, n_active, block_expert, nb):
    p_total = xs.shape[0]
    grid_spec = pltpu.PrefetchScalarGridSpec(
        num_scalar_prefetch=2,
        grid=(nb,),
        in_specs=[
            pl.BlockSpec((BN, H), lambda i, na, be: (i, 0)),
            pl.BlockSpec((1, H, FF), lambda i, na, be: (be[i], 0, 0)),
            pl.BlockSpec((1, 1, FF), lambda i, na, be: (be[i], 0, 0)),
            pl.BlockSpec((1, FF, H), lambda i, na, be: (be[i], 0, 0)),
            pl.BlockSpec((1, 1, H), lambda i, na, be: (be[i], 0, 0)),
        ],
        out_specs=pl.BlockSpec((BN, H), lambda i, na, be: (i, 0)),
    )
    return pl.pallas_call(
        _ffn_body,
        grid_spec=grid_spec,
        out_shape=jax.ShapeDtypeStruct((p_total, H), jnp.float32),
        )(n_active, block_expert, xs,
      fc1_w.astype(jnp.bfloat16), fc1_b.reshape(E, 1, FF),
      fc2_w.astype(jnp.bfloat16), fc2_b.reshape(E, 1, H))


def kernel(hidden_states, router_w, router_b, fc1_w, fc1_b, fc2_w, fc2_b):
    b, s, h = hidden_states.shape
    n = b * s
    x2d = hidden_states.reshape(n, h)

    top2, loss, xcp = _run_router(x2d, router_w, router_b)
    (gather_idx, block_expert, n_active,
     pos_a, pos_b, p_total, nb) = _dispatch_plan(top2, n)

    g_chunk, c_chunk = 32, 16
    sc_gather = _make_sc_gather(p_total, h, g_chunk, jnp.float32)
    xs = sc_gather(xcp, gather_idx.reshape(NW, -1, g_chunk))

    ys = _run_ffn(xs, fc1_w, fc1_b, fc2_w, fc2_b, n_active, block_expert, nb)

    sc_combine = _make_sc_combine(n, h, c_chunk)
    out = sc_combine(ys, pos_a.reshape(NW, -1, c_chunk),
                     pos_b.reshape(NW, -1, c_chunk))
    return out.reshape(b, s, h), loss


# speedup vs baseline: 1.0072x; 1.0072x over previous
import functools

import jax
import jax.numpy as jnp
from jax import lax
from jax.experimental import pallas as pl
from jax.experimental.pallas import tpu as pltpu
from jax.experimental.pallas import tpu_sc as plsc

E = 8
K = 2
H = 1024
FF = 4096
BN = 256

NC = 2
NS = 16
NW = NC * NS


def _router_body(x_ref, w_ref, b_ref, idx_ref, loss_ref):
    x = x_ref[...]
    logits = jnp.dot(x, w_ref[...], preferred_element_type=jnp.float32)
    logits = logits + b_ref[...]
    z = logits - jnp.max(logits, axis=1, keepdims=True)
    ez = jnp.exp(z)
    p = ez / jnp.sum(ez, axis=1, keepdims=True)

    iota = lax.broadcasted_iota(jnp.int32, p.shape, 1)
    m1 = jnp.max(p, axis=1, keepdims=True)
    i1 = jnp.min(jnp.where(p >= m1, iota, E), axis=1, keepdims=True)
    p2 = jnp.where(iota == i1, -jnp.float32(jnp.inf), p)
    m2 = jnp.max(p2, axis=1, keepdims=True)
    i2 = jnp.min(jnp.where(p2 >= m2, iota, E), axis=1, keepdims=True)
    idx_ref[...] = jnp.concatenate([i1, i2], axis=1)

    imp = jnp.mean(p, axis=0, keepdims=True)
    loss_ref[...] = 0.01 * jnp.sum(E * imp * jnp.log(imp)).reshape(1, 1)


def _run_router(x2d, router_w, router_b):
    n, h = x2d.shape
    top2, loss = pl.pallas_call(
        _router_body,
        out_shape=(
            jax.ShapeDtypeStruct((n, K), jnp.int32),
            jax.ShapeDtypeStruct((1, 1), jnp.float32),
        ),
    )(x2d, router_w, router_b.reshape(1, E))
    return top2, loss[0, 0]


def _dispatch_plan(top2, n_tokens):
    nk = n_tokens * K
    p_total = nk + E * BN
    nb = p_total // BN

    e_flat = top2.reshape(-1).astype(jnp.int32)
    onehot = (e_flat[:, None] == jnp.arange(E, dtype=jnp.int32)[None, :])
    inc = jnp.cumsum(onehot.astype(jnp.int32), axis=0)
    rank = jnp.take_along_axis(inc, e_flat[:, None], axis=1)[:, 0] - 1
    counts = inc[-1]
    padded = ((counts + BN - 1) // BN) * BN
    pstart = jnp.concatenate(
        [jnp.zeros((1,), jnp.int32), jnp.cumsum(padded)[:-1].astype(jnp.int32)])
    pos = pstart[e_flat] + rank

    token_of_entry = (jnp.arange(nk, dtype=jnp.int32) // K)
    gather_idx = jnp.zeros((p_total,), jnp.int32).at[pos].set(token_of_entry)

    pend = (pstart + padded).astype(jnp.int32)
    blk = jnp.arange(nb, dtype=jnp.int32) * BN
    block_expert = jnp.sum((blk[:, None] >= pend[None, :]).astype(jnp.int32),
                           axis=1)
    block_expert = jnp.clip(block_expert, 0, E - 1)
    n_active = (jnp.sum(padded) // BN).astype(jnp.int32).reshape(1)

    pos_a = pos[0::K]
    pos_b = pos[1::K]
    return gather_idx, block_expert, n_active, pos_a, pos_b, p_total, nb


def _make_sc_gather(n_rows_out, d, chunk, dtype):
    rows_per_w = n_rows_out // NW
    n_chunks = rows_per_w // chunk
    mesh = plsc.VectorSubcoreMesh(core_axis_name="c", subcore_axis_name="s")

    @functools.partial(
        pl.kernel,
        mesh=mesh,
        out_type=jax.ShapeDtypeStruct((n_rows_out, d), dtype),
        scratch_types=[
            pltpu.VMEM((n_chunks, chunk), jnp.int32),
            pltpu.VMEM((chunk, d), dtype),
            pltpu.VMEM((chunk, d), dtype),
            pltpu.SemaphoreType.DMA,
            pltpu.SemaphoreType.DMA,
            pltpu.SemaphoreType.DMA,
            pltpu.SemaphoreType.DMA,
        ],
    )
    def gather(table_hbm, idx_hbm, out_hbm, idx_v, buf0, buf1,
               gs0, gs1, os0, os1):
        wid = lax.axis_index("s") * NC + lax.axis_index("c")
        base = wid * rows_per_w
        pltpu.sync_copy(idx_hbm.at[wid], idx_v)
        bufs = (buf0, buf1)
        gsems = (gs0, gs1)
        osems = (os0, os1)
        ocopies = [None, None]
        gcopies = [None, None]
        gcopies[0] = pltpu.async_copy(table_hbm.at[idx_v.at[0]], buf0, gs0)
        for c in range(n_chunks):
            p = c % 2
            q = (c + 1) % 2
            if c + 1 < n_chunks:
                if c >= 1:
                    ocopies[q].wait()
                gcopies[q] = pltpu.async_copy(
                    table_hbm.at[idx_v.at[c + 1]], bufs[q], gsems[q])
            gcopies[p].wait()
            ocopies[p] = pltpu.async_copy(
                bufs[p], out_hbm.at[pl.ds(base + c * chunk, chunk)], osems[p])
        for c in range(max(0, n_chunks - 2), n_chunks):
            ocopies[c % 2].wait()

    return gather


def _make_sc_combine(n_tokens, d, chunk):
    rows_per_w = n_tokens // NW
    n_chunks = rows_per_w // chunk
    mesh = plsc.VectorSubcoreMesh(core_axis_name="c", subcore_axis_name="s")
    lanes_per_row = d // 16

    @functools.partial(
        pl.kernel,
        mesh=mesh,
        out_type=jax.ShapeDtypeStruct((n_tokens, d), jnp.float32),
        scratch_types=[
            pltpu.VMEM((n_chunks, chunk), jnp.int32),
            pltpu.VMEM((n_chunks, chunk), jnp.int32),
            pltpu.VMEM((chunk, d), jnp.float32),
            pltpu.VMEM((chunk, d), jnp.float32),
            pltpu.VMEM((chunk, d), jnp.float32),
            pltpu.VMEM((chunk, d), jnp.float32),
            pltpu.SemaphoreType.DMA,
            pltpu.SemaphoreType.DMA,
            pltpu.SemaphoreType.DMA,
            pltpu.SemaphoreType.DMA,
        ],
    )
    def combine(y_hbm, ia_hbm, ib_hbm, out_hbm,
                ia_v, ib_v, ra0, ra1, rb0, rb1, gs0, gs1, os0, os1):
        wid = lax.axis_index("s") * NC + lax.axis_index("c")
        base = wid * rows_per_w
        pltpu.sync_copy(ia_hbm.at[wid], ia_v)
        pltpu.sync_copy(ib_hbm.at[wid], ib_v)
        ras = (ra0, ra1)
        rbs = (rb0, rb1)
        gsems = (gs0, gs1)
        osems = (os0, os1)
        ocopies = [None, None]
        gcopies = [None, None]
        gcopies[0] = (pltpu.async_copy(y_hbm.at[ia_v.at[0]], ra0, gs0),
                      pltpu.async_copy(y_hbm.at[ib_v.at[0]], rb0, gs0))
        for c in range(n_chunks):
            p = c % 2
            q = (c + 1) % 2
            if c + 1 < n_chunks:
                if c >= 1:
                    ocopies[q].wait()
                gcopies[q] = (
                    pltpu.async_copy(y_hbm.at[ia_v.at[c + 1]], ras[q], gsems[q]),
                    pltpu.async_copy(y_hbm.at[ib_v.at[c + 1]], rbs[q], gsems[q]))
            gcopies[p][0].wait()
            gcopies[p][1].wait()

            def row_add(r, carry, _ra=ras[p], _rb=rbs[p]):
                for s in range(lanes_per_row):
                    a = _ra[r, pl.ds(s * 16, 16)]
                    b = _rb[r, pl.ds(s * 16, 16)]
                    _ra[r, pl.ds(s * 16, 16)] = a + b
                return carry

            lax.fori_loop(0, chunk, row_add, 0)
            ocopies[p] = pltpu.async_copy(
                ras[p], out_hbm.at[pl.ds(base + c * chunk, chunk)], osems[p])
        for c in range(max(0, n_chunks - 2), n_chunks):
            ocopies[c % 2].wait()

    return combine


def _ffn_body(nact_ref, be_ref, x_ref, w1_ref, b1_ref, w2_ref, b2_ref, o_ref):
    i = pl.program_id(0)

    @pl.when(i < nact_ref[0])
    def _():
        x = x_ref[...].astype(jnp.bfloat16)
        h = jnp.dot(x, w1_ref[0], preferred_element_type=jnp.float32)
        h = h + b1_ref[0]
        h = h * 0.5 * (1.0 + lax.erf(h * 0.7071067811865476))
        y = jnp.dot(h.astype(jnp.bfloat16), w2_ref[0],
                    preferred_element_type=jnp.float32)
        o_ref[...] = y + b2_ref[0]


def _run_ffn(xs, fc1_wb, fc1_b, fc2_wb, fc2_b, n_active, block_expert, nb):
    p_total = xs.shape[0]
    grid_spec = pltpu.PrefetchScalarGridSpec(
        num_scalar_prefetch=2,
        grid=(nb,),
        in_specs=[
            pl.BlockSpec((BN, H), lambda i, na, be: (i, 0)),
            pl.BlockSpec((1, H, FF), lambda i, na, be: (be[i], 0, 0)),
            pl.BlockSpec((1, 1, FF), lambda i, na, be: (be[i], 0, 0)),
            pl.BlockSpec((1, FF, H), lambda i, na, be: (be[i], 0, 0)),
            pl.BlockSpec((1, 1, H), lambda i, na, be: (be[i], 0, 0)),
        ],
        out_specs=pl.BlockSpec((BN, H), lambda i, na, be: (i, 0)),
    )
    return pl.pallas_call(
        _ffn_body,
        grid_spec=grid_spec,
        out_shape=jax.ShapeDtypeStruct((p_total, H), jnp.float32),
    )(n_active, block_expert, xs,
      fc1_wb, fc1_b.reshape(E, 1, FF),
      fc2_wb, fc2_b.reshape(E, 1, H))


def kernel(hidden_states, router_w, router_b, fc1_w, fc1_b, fc2_w, fc2_b):
    b, s, h = hidden_states.shape
    n = b * s
    x2d = hidden_states.reshape(n, h)

    top2, loss = _run_router(x2d, router_w, router_b)
    (gather_idx, block_expert, n_active,
     pos_a, pos_b, p_total, nb) = _dispatch_plan(top2, n)

    fc1_wb = fc1_w.astype(jnp.bfloat16)
    fc2_wb = fc2_w.astype(jnp.bfloat16)
    fence = (fc1_wb[0, 0, 0].astype(jnp.int32) * 0 +
             fc2_wb[0, 0, 0].astype(jnp.int32) * 0)
    gather_idx = gather_idx + fence

    g_chunk, c_chunk = 32, 16
    sc_gather = _make_sc_gather(p_total, h, g_chunk, jnp.float32)
    xs = sc_gather(x2d, gather_idx.reshape(NW, -1, g_chunk))

    ys = _run_ffn(xs, fc1_wb, fc1_b, fc2_wb, fc2_b, n_active, block_expert, nb)

    sc_combine = _make_sc_combine(n, h, c_chunk)
    out = sc_combine(ys, pos_a.reshape(NW, -1, c_chunk),
                     pos_b.reshape(NW, -1, c_chunk))
    return out.reshape(b, s, h), loss


# speedup vs baseline: 1.0169x; 1.0097x over previous
import functools

import jax
import jax.numpy as jnp
from jax import lax
from jax.experimental import pallas as pl
from jax.experimental.pallas import tpu as pltpu
from jax.experimental.pallas import tpu_sc as plsc

E = 8
K = 2
H = 1024
FF = 4096
BN = 256

NC = 2
NS = 16
NW = NC * NS


def _router_body(x_ref, w_ref, b_ref, idx_ref, loss_ref):
    x = x_ref[...]
    logits = jnp.dot(x, w_ref[...], preferred_element_type=jnp.float32)
    logits = logits + b_ref[...]
    z = logits - jnp.max(logits, axis=1, keepdims=True)
    ez = jnp.exp(z)
    p = ez / jnp.sum(ez, axis=1, keepdims=True)

    iota = lax.broadcasted_iota(jnp.int32, p.shape, 1)
    m1 = jnp.max(p, axis=1, keepdims=True)
    i1 = jnp.min(jnp.where(p >= m1, iota, E), axis=1, keepdims=True)
    p2 = jnp.where(iota == i1, -jnp.float32(jnp.inf), p)
    m2 = jnp.max(p2, axis=1, keepdims=True)
    i2 = jnp.min(jnp.where(p2 >= m2, iota, E), axis=1, keepdims=True)
    idx_ref[...] = jnp.concatenate([i1, i2], axis=1)

    imp = jnp.mean(p, axis=0, keepdims=True)
    loss_ref[...] = 0.01 * jnp.sum(E * imp * jnp.log(imp)).reshape(1, 1)


def _run_router(x2d, router_w, router_b):
    n, h = x2d.shape
    top2, loss = pl.pallas_call(
        _router_body,
        out_shape=(
            jax.ShapeDtypeStruct((n, K), jnp.int32),
            jax.ShapeDtypeStruct((1, 1), jnp.float32),
        ),
    )(x2d, router_w, router_b.reshape(1, E))
    return top2, loss[0, 0]


def _dispatch_plan(top2, n_tokens):
    nk = n_tokens * K
    p_total = nk + E * BN
    nb = p_total // BN

    e_flat = top2.reshape(-1).astype(jnp.int32)
    onehot = (e_flat[:, None] == jnp.arange(E, dtype=jnp.int32)[None, :])
    inc = jnp.cumsum(onehot.astype(jnp.int32), axis=0)
    rank = jnp.take_along_axis(inc, e_flat[:, None], axis=1)[:, 0] - 1
    counts = inc[-1]
    padded = ((counts + BN - 1) // BN) * BN
    pstart = jnp.concatenate(
        [jnp.zeros((1,), jnp.int32), jnp.cumsum(padded)[:-1].astype(jnp.int32)])
    pos = pstart[e_flat] + rank

    token_of_entry = (jnp.arange(nk, dtype=jnp.int32) // K)
    gather_idx = jnp.zeros((p_total,), jnp.int32).at[pos].set(token_of_entry)

    pend = (pstart + padded).astype(jnp.int32)
    blk = jnp.arange(nb, dtype=jnp.int32) * BN
    block_expert = jnp.sum((blk[:, None] >= pend[None, :]).astype(jnp.int32),
                           axis=1)
    block_expert = jnp.clip(block_expert, 0, E - 1)
    n_active = (jnp.sum(padded) // BN).astype(jnp.int32).reshape(1)

    pos_a = pos[0::K]
    pos_b = pos[1::K]
    return gather_idx, block_expert, n_active, pos_a, pos_b, p_total, nb


def _make_sc_gather(n_rows_out, d, chunk, dtype):
    rows_per_w = n_rows_out // NW
    n_chunks = rows_per_w // chunk
    mesh = plsc.VectorSubcoreMesh(core_axis_name="c", subcore_axis_name="s")

    @functools.partial(
        pl.kernel,
        mesh=mesh,
        out_type=jax.ShapeDtypeStruct((n_rows_out, d), dtype),
        scratch_types=[
            pltpu.VMEM((n_chunks, chunk), jnp.int32),
            pltpu.VMEM((chunk, d), dtype),
            pltpu.VMEM((chunk, d), dtype),
            pltpu.SemaphoreType.DMA,
            pltpu.SemaphoreType.DMA,
            pltpu.SemaphoreType.DMA,
            pltpu.SemaphoreType.DMA,
        ],
    )
    def gather(table_hbm, idx_hbm, out_hbm, idx_v, buf0, buf1,
               gs0, gs1, os0, os1):
        wid = lax.axis_index("s") * NC + lax.axis_index("c")
        base = wid * rows_per_w
        pltpu.sync_copy(idx_hbm.at[wid], idx_v)
        bufs = (buf0, buf1)
        gsems = (gs0, gs1)
        osems = (os0, os1)
        ocopies = [None, None]
        gcopies = [None, None]
        gcopies[0] = pltpu.async_copy(table_hbm.at[idx_v.at[0]], buf0, gs0)
        for c in range(n_chunks):
            p = c % 2
            q = (c + 1) % 2
            if c + 1 < n_chunks:
                if c >= 1:
                    ocopies[q].wait()
                gcopies[q] = pltpu.async_copy(
                    table_hbm.at[idx_v.at[c + 1]], bufs[q], gsems[q])
            gcopies[p].wait()
            ocopies[p] = pltpu.async_copy(
                bufs[p], out_hbm.at[pl.ds(base + c * chunk, chunk)], osems[p])
        for c in range(max(0, n_chunks - 2), n_chunks):
            ocopies[c % 2].wait()

    return gather


def _make_sc_combine(n_tokens, d, chunk):
    rows_per_w = n_tokens // NW
    n_chunks = rows_per_w // chunk
    mesh = plsc.VectorSubcoreMesh(core_axis_name="c", subcore_axis_name="s")
    lanes_per_row = d // 16

    @functools.partial(
        pl.kernel,
        mesh=mesh,
        out_type=jax.ShapeDtypeStruct((n_tokens, d), jnp.float32),
        scratch_types=[
            pltpu.VMEM((n_chunks, chunk), jnp.int32),
            pltpu.VMEM((n_chunks, chunk), jnp.int32),
            pltpu.VMEM((chunk, d), jnp.float32),
            pltpu.VMEM((chunk, d), jnp.float32),
            pltpu.VMEM((chunk, d), jnp.float32),
            pltpu.VMEM((chunk, d), jnp.float32),
            pltpu.SemaphoreType.DMA,
            pltpu.SemaphoreType.DMA,
            pltpu.SemaphoreType.DMA,
            pltpu.SemaphoreType.DMA,
        ],
    )
    def combine(y_hbm, ia_hbm, ib_hbm, out_hbm,
                ia_v, ib_v, ra0, ra1, rb0, rb1, gs0, gs1, os0, os1):
        wid = lax.axis_index("s") * NC + lax.axis_index("c")
        base = wid * rows_per_w
        pltpu.sync_copy(ia_hbm.at[wid], ia_v)
        pltpu.sync_copy(ib_hbm.at[wid], ib_v)
        ras = (ra0, ra1)
        rbs = (rb0, rb1)
        gsems = (gs0, gs1)
        osems = (os0, os1)
        ocopies = [None, None]
        gcopies = [None, None]
        gcopies[0] = (pltpu.async_copy(y_hbm.at[ia_v.at[0]], ra0, gs0),
                      pltpu.async_copy(y_hbm.at[ib_v.at[0]], rb0, gs0))
        for c in range(n_chunks):
            p = c % 2
            q = (c + 1) % 2
            if c + 1 < n_chunks:
                if c >= 1:
                    ocopies[q].wait()
                gcopies[q] = (
                    pltpu.async_copy(y_hbm.at[ia_v.at[c + 1]], ras[q], gsems[q]),
                    pltpu.async_copy(y_hbm.at[ib_v.at[c + 1]], rbs[q], gsems[q]))
            gcopies[p][0].wait()
            gcopies[p][1].wait()

            def row_add(r, carry, _ra=ras[p], _rb=rbs[p]):
                for s in range(lanes_per_row):
                    a = _ra[r, pl.ds(s * 16, 16)]
                    b = _rb[r, pl.ds(s * 16, 16)]
                    _ra[r, pl.ds(s * 16, 16)] = a + b
                return carry

            lax.fori_loop(0, chunk, row_add, 0)
            ocopies[p] = pltpu.async_copy(
                ras[p], out_hbm.at[pl.ds(base + c * chunk, chunk)], osems[p])
        for c in range(max(0, n_chunks - 2), n_chunks):
            ocopies[c % 2].wait()

    return combine


def _ffn_body(nact_ref, be_ref, x_ref, w1_ref, b1_ref, w2_ref, b2_ref, o_ref):
    i = pl.program_id(0)

    @pl.when(i < nact_ref[0])
    def _():
        x = x_ref[...].astype(jnp.bfloat16)
        h = jnp.dot(x, w1_ref[0], preferred_element_type=jnp.float32)
        h = h + b1_ref[0]
        h = h * 0.5 * (1.0 + lax.erf(h * 0.7071067811865476))
        y = jnp.dot(h.astype(jnp.bfloat16), w2_ref[0],
                    preferred_element_type=jnp.float32)
        o_ref[...] = y + b2_ref[0]


def _run_ffn(xs, fc1_wb, fc1_b, fc2_wb, fc2_b, n_active, block_expert, nb):
    p_total = xs.shape[0]
    grid_spec = pltpu.PrefetchScalarGridSpec(
        num_scalar_prefetch=2,
        grid=(nb,),
        in_specs=[
            pl.BlockSpec((BN, H), lambda i, na, be: (i, 0)),
            pl.BlockSpec((1, H, FF), lambda i, na, be: (be[i], 0, 0)),
            pl.BlockSpec((1, 1, FF), lambda i, na, be: (be[i], 0, 0)),
            pl.BlockSpec((1, FF, H), lambda i, na, be: (be[i], 0, 0)),
            pl.BlockSpec((1, 1, H), lambda i, na, be: (be[i], 0, 0)),
        ],
        out_specs=pl.BlockSpec((BN, H), lambda i, na, be: (i, 0)),
    )
    return pl.pallas_call(
        _ffn_body,
        grid_spec=grid_spec,
        out_shape=jax.ShapeDtypeStruct((p_total, H), jnp.float32),
    )(n_active, block_expert, xs,
      fc1_wb, fc1_b.reshape(E, 1, FF),
      fc2_wb, fc2_b.reshape(E, 1, H))


def kernel(hidden_states, router_w, router_b, fc1_w, fc1_b, fc2_w, fc2_b):
    b, s, h = hidden_states.shape
    n = b * s
    x2d = hidden_states.reshape(n, h)

    top2, loss = _run_router(x2d, router_w, router_b)
    (gather_idx, block_expert, n_active,
     pos_a, pos_b, p_total, nb) = _dispatch_plan(top2, n)

    fc1_wb = fc1_w.astype(jnp.bfloat16)
    fc2_wb = fc2_w.astype(jnp.bfloat16)
    gather_idx, fc1_wb, fc2_wb = lax.optimization_barrier(
        (gather_idx, fc1_wb, fc2_wb))

    g_chunk, c_chunk = 32, 16
    sc_gather = _make_sc_gather(p_total, h, g_chunk, jnp.float32)
    xs = sc_gather(x2d, gather_idx.reshape(NW, -1, g_chunk))

    ys = _run_ffn(xs, fc1_wb, fc1_b, fc2_wb, fc2_b, n_active, block_expert, nb)

    sc_combine = _make_sc_combine(n, h, c_chunk)
    out = sc_combine(ys, pos_a.reshape(NW, -1, c_chunk),
                     pos_b.reshape(NW, -1, c_chunk))
    return out.reshape(b, s, h), loss
